# baseline (speedup 1.0000x reference)
import jax
import jax.numpy as jnp
from jax import lax
from jax.experimental import pallas as pl
from jax.experimental.pallas import tpu as pltpu

B = 1024
D = 768
H = 2 * D
SROWS = 800


def _mega_body(eidx_sref, sidx_sref,
               bert_ref, aux_ref, w1_ref, w3_ref,
               b1_ref, w2t_ref, b3_ref, w4t_ref, b2_ref, b4_ref,
               emb_ref, prof_ref, out_ref,
               ebuf, pbuf, lbuf, w1buf, w3buf,
               sem_g, sem_p, sem_w, sem_o):
    cp_l = pltpu.make_async_copy(aux_ref.at[pl.ds(SROWS, B)], lbuf, sem_o)
    cp_l.start()
    cp_w1 = pltpu.make_async_copy(w1_ref, w1buf, sem_w)
    cp_w1.start()
    cp_w3 = pltpu.make_async_copy(w3_ref, w3buf, sem_w)
    cp_w3.start()

    def issue_e(j, _):
        pltpu.make_async_copy(
            bert_ref.at[pl.ds(eidx_sref[j], 1)], ebuf.at[pl.ds(j, 1)], sem_g
        ).start()
        return 0
    lax.fori_loop(0, B, issue_e, 0, unroll=16)

    def issue_p(j, _):
        pltpu.make_async_copy(
            aux_ref.at[pl.ds(sidx_sref[j] // 128, 1)], pbuf.at[pl.ds(j, 1)],
            sem_p
        ).start()
        return 0
    lax.fori_loop(0, B, issue_p, 0, unroll=16)

    cp_w1.wait()
    cp_w3.wait()
    cp_l.wait()
    pltpu.make_async_copy(bert_ref.at[pl.ds(0, B)], ebuf, sem_g).wait()
    pltpu.make_async_copy(aux_ref.at[pl.ds(0, B)], pbuf, sem_p).wait()

    x = ebuf[...]
    cp_e = pltpu.make_async_copy(ebuf, emb_ref, sem_o)
    cp_e.start()
    h1 = jax.nn.sigmoid(
        jnp.dot(x, w1buf[...], preferred_element_type=jnp.float32)
        + b1_ref[...])
    a = jax.nn.sigmoid(
        jnp.sum(h1 * w2t_ref[...], axis=1, keepdims=True) + b2_ref[0, 0])
    h2 = jax.nn.sigmoid(
        jnp.dot(x, w3buf[...], preferred_element_type=jnp.float32)
        + b3_ref[...])
    bb = jnp.sum(h2 * w4t_ref[...], axis=1, keepdims=True) + b4_ref[0, 0]
    pcol = jnp.sum(pbuf[...] * lbuf[...], axis=1, keepdims=True)
    a8 = jnp.reshape(a, (8, 128))
    b8 = jnp.reshape(bb, (8, 128))
    p8 = jnp.reshape(pcol, (8, 128))
    prof_ref[...] = p8
    out_ref[...] = jax.nn.sigmoid(1.703 * a8 * (p8 - b8))
    cp_e.wait()


def kernel(stu_ids, exer_in, bert_table, stu_table,
           W_disc1, b_disc1, W_disc2, b_disc2,
           W_diff1, b_diff1, W_diff2, b_diff2):
    grid_spec = pltpu.PrefetchScalarGridSpec(
        num_scalar_prefetch=2,
        grid=(1,),
        in_specs=[
            pl.BlockSpec(memory_space=pl.ANY),
            pl.BlockSpec(memory_space=pl.ANY),
            pl.BlockSpec(memory_space=pl.ANY),
            pl.BlockSpec(memory_space=pl.ANY),
            pl.BlockSpec((1, H), lambda i, *_: (0, 0)),
            pl.BlockSpec((1, H), lambda i, *_: (0, 0)),
            pl.BlockSpec((1, D), lambda i, *_: (0, 0)),
            pl.BlockSpec((1, D), lambda i, *_: (0, 0)),
            pl.BlockSpec(memory_space=pltpu.SMEM),
            pl.BlockSpec(memory_space=pltpu.SMEM),
        ],
        out_specs=[
            pl.BlockSpec(memory_space=pl.ANY),
            pl.BlockSpec((8, 128), lambda i, *_: (0, 0)),
            pl.BlockSpec((8, 128), lambda i, *_: (0, 0)),
        ],
        scratch_shapes=[
            pltpu.VMEM((B, D), jnp.float32),
            pltpu.VMEM((B, 128), jnp.float32),
            pltpu.VMEM((B, 128), jnp.float32),
            pltpu.VMEM((D, H), jnp.float32),
            pltpu.VMEM((D, D), jnp.float32),
            pltpu.SemaphoreType.DMA,
            pltpu.SemaphoreType.DMA,
            pltpu.SemaphoreType.DMA,
            pltpu.SemaphoreType.DMA,
        ],
    )
    sids32 = stu_ids.astype(jnp.int32)
    stu_rows = jnp.concatenate(
        [stu_table.reshape(-1),
         jnp.zeros((SROWS * 128 - 100000,), jnp.float32)]).reshape(SROWS, 128)
    onehot = (jnp.arange(128, dtype=jnp.int32)[None, :]
              == (sids32 & 127)[:, None]).astype(jnp.float32)
    aux = jnp.concatenate([stu_rows, onehot], axis=0)
    emb, prof, outc = pl.pallas_call(
        _mega_body,
        grid_spec=grid_spec,
        out_shape=[
            jax.ShapeDtypeStruct((B, D), jnp.float32),
            jax.ShapeDtypeStruct((8, 128), jnp.float32),
            jax.ShapeDtypeStruct((8, 128), jnp.float32),
        ],
    )(exer_in.astype(jnp.int32), sids32,
      bert_table, aux, W_disc1, W_diff1,
      b_disc1.reshape(1, H), W_disc2.reshape(1, H),
      b_diff1.reshape(1, D), W_diff2.reshape(1, D),
      b_disc2.reshape(1, 1), b_diff2.reshape(1, 1))
    return (outc.reshape(B), emb, prof.reshape(B, 1))


# speedup vs baseline: 1.0057x; 1.0057x over previous
import jax
import jax.numpy as jnp
from jax import lax
from jax.experimental import pallas as pl
from jax.experimental.pallas import tpu as pltpu

B = 1024
D = 768
H = 2 * D
SROWS = 800


def _mega_body(eidx_sref, sidx_sref,
               bert_ref, aux_ref, w1_ref, w3_ref,
               b1_ref, w2t_ref, b3_ref, w4t_ref, b2_ref, b4_ref,
               emb_ref, prof_ref, out_ref,
               ebuf, pbuf, lbuf, w1buf, w3buf,
               sem_g, sem_g2, sem_p, sem_w, sem_o):
    cp_l = pltpu.make_async_copy(aux_ref.at[pl.ds(SROWS, B)], lbuf, sem_o)
    cp_l.start()
    cp_w1 = pltpu.make_async_copy(w1_ref, w1buf, sem_w)
    cp_w1.start()
    cp_w3 = pltpu.make_async_copy(w3_ref, w3buf, sem_w)
    cp_w3.start()

    def issue_e(j, _):
        pltpu.make_async_copy(
            bert_ref.at[pl.ds(eidx_sref[2 * j], 1)],
            ebuf.at[pl.ds(2 * j, 1)], sem_g
        ).start()
        pltpu.make_async_copy(
            bert_ref.at[pl.ds(eidx_sref[2 * j + 1], 1)],
            ebuf.at[pl.ds(2 * j + 1, 1)], sem_g2
        ).start()
        return 0
    lax.fori_loop(0, B // 2, issue_e, 0, unroll=8)

    def issue_p(j, _):
        pltpu.make_async_copy(
            aux_ref.at[pl.ds(sidx_sref[j] // 128, 1)], pbuf.at[pl.ds(j, 1)],
            sem_p
        ).start()
        return 0
    lax.fori_loop(0, B, issue_p, 0, unroll=16)

    cp_w1.wait()
    cp_w3.wait()
    cp_l.wait()
    pltpu.make_async_copy(bert_ref.at[pl.ds(0, B // 2)],
                          ebuf.at[pl.ds(0, B // 2)], sem_g).wait()
    pltpu.make_async_copy(bert_ref.at[pl.ds(0, B // 2)],
                          ebuf.at[pl.ds(0, B // 2)], sem_g2).wait()
    pltpu.make_async_copy(aux_ref.at[pl.ds(0, B)], pbuf, sem_p).wait()

    x = ebuf[...]
    cp_e = pltpu.make_async_copy(ebuf, emb_ref, sem_o)
    cp_e.start()
    h1 = jax.nn.sigmoid(
        jnp.dot(x, w1buf[...], preferred_element_type=jnp.float32)
        + b1_ref[...])
    a = jax.nn.sigmoid(
        jnp.sum(h1 * w2t_ref[...], axis=1, keepdims=True) + b2_ref[0, 0])
    h2 = jax.nn.sigmoid(
        jnp.dot(x, w3buf[...], preferred_element_type=jnp.float32)
        + b3_ref[...])
    bb = jnp.sum(h2 * w4t_ref[...], axis=1, keepdims=True) + b4_ref[0, 0]
    pcol = jnp.sum(pbuf[...] * lbuf[...], axis=1, keepdims=True)
    a8 = jnp.reshape(a, (8, 128))
    b8 = jnp.reshape(bb, (8, 128))
    p8 = jnp.reshape(pcol, (8, 128))
    prof_ref[...] = p8
    out_ref[...] = jax.nn.sigmoid(1.703 * a8 * (p8 - b8))
    cp_e.wait()


def kernel(stu_ids, exer_in, bert_table, stu_table,
           W_disc1, b_disc1, W_disc2, b_disc2,
           W_diff1, b_diff1, W_diff2, b_diff2):
    grid_spec = pltpu.PrefetchScalarGridSpec(
        num_scalar_prefetch=2,
        grid=(1,),
        in_specs=[
            pl.BlockSpec(memory_space=pl.ANY),
            pl.BlockSpec(memory_space=pl.ANY),
            pl.BlockSpec(memory_space=pl.ANY),
            pl.BlockSpec(memory_space=pl.ANY),
            pl.BlockSpec((1, H), lambda i, *_: (0, 0)),
            pl.BlockSpec((1, H), lambda i, *_: (0, 0)),
            pl.BlockSpec((1, D), lambda i, *_: (0, 0)),
            pl.BlockSpec((1, D), lambda i, *_: (0, 0)),
            pl.BlockSpec(memory_space=pltpu.SMEM),
            pl.BlockSpec(memory_space=pltpu.SMEM),
        ],
        out_specs=[
            pl.BlockSpec(memory_space=pl.ANY),
            pl.BlockSpec((8, 128), lambda i, *_: (0, 0)),
            pl.BlockSpec((8, 128), lambda i, *_: (0, 0)),
        ],
        scratch_shapes=[
            pltpu.VMEM((B, D), jnp.float32),
            pltpu.VMEM((B, 128), jnp.float32),
            pltpu.VMEM((B, 128), jnp.float32),
            pltpu.VMEM((D, H), jnp.float32),
            pltpu.VMEM((D, D), jnp.float32),
            pltpu.SemaphoreType.DMA,
            pltpu.SemaphoreType.DMA,
            pltpu.SemaphoreType.DMA,
            pltpu.SemaphoreType.DMA,
            pltpu.SemaphoreType.DMA,
        ],
    )
    sids32 = stu_ids.astype(jnp.int32)
    stu_rows = jnp.concatenate(
        [stu_table.reshape(-1),
         jnp.zeros((SROWS * 128 - 100000,), jnp.float32)]).reshape(SROWS, 128)
    onehot = (jnp.arange(128, dtype=jnp.int32)[None, :]
              == (sids32 & 127)[:, None]).astype(jnp.float32)
    aux = jnp.concatenate([stu_rows, onehot], axis=0)
    emb, prof, outc = pl.pallas_call(
        _mega_body,
        grid_spec=grid_spec,
        out_shape=[
            jax.ShapeDtypeStruct((B, D), jnp.float32),
            jax.ShapeDtypeStruct((8, 128), jnp.float32),
            jax.ShapeDtypeStruct((8, 128), jnp.float32),
        ],
    )(exer_in.astype(jnp.int32), sids32,
      bert_table, aux, W_disc1, W_diff1,
      b_disc1.reshape(1, H), W_disc2.reshape(1, H),
      b_diff1.reshape(1, D), W_diff2.reshape(1, D),
      b_disc2.reshape(1, 1), b_diff2.reshape(1, 1))
    return (outc.reshape(B), emb, prof.reshape(B, 1))


# speedup vs baseline: 1.0182x; 1.0124x over previous
import jax
import jax.numpy as jnp
from jax import lax
from jax.experimental import pallas as pl
from jax.experimental.pallas import tpu as pltpu

B = 1024
D = 768
H = 2 * D
SROWS = 800


def _mega_body(eidx_sref, sidx_sref,
               bert_ref, aux_ref, w1_ref, w3_ref,
               b1_ref, w2t_ref, b3_ref, w4t_ref, b2_ref, b4_ref,
               emb_ref, prof_ref, out_ref,
               ebuf, pbuf, lbuf, w1buf, w3buf,
               sem_g, sem_g2, sem_p, sem_w, sem_o):
    cp_l = pltpu.make_async_copy(aux_ref.at[pl.ds(SROWS, B)], lbuf, sem_o)
    cp_l.start()
    cp_w1 = pltpu.make_async_copy(w1_ref, w1buf, sem_w)
    cp_w1.start()
    cp_w3 = pltpu.make_async_copy(w3_ref, w3buf, sem_w)
    cp_w3.start()

    def issue_e(j, _):
        pltpu.make_async_copy(
            bert_ref.at[pl.ds(eidx_sref[2 * j], 1)],
            ebuf.at[pl.ds(2 * j, 1)], sem_g
        ).start()
        pltpu.make_async_copy(
            bert_ref.at[pl.ds(eidx_sref[2 * j + 1], 1)],
            ebuf.at[pl.ds(2 * j + 1, 1)], sem_g2
        ).start()
        return 0
    lax.fori_loop(0, B // 2, issue_e, 0, unroll=8)

    def issue_p(j, _):
        pltpu.make_async_copy(
            aux_ref.at[pl.ds(sidx_sref[j] // 128, 1)], pbuf.at[pl.ds(j, 1)],
            sem_p
        ).start()
        return 0
    lax.fori_loop(0, B, issue_p, 0, unroll=16)

    cp_w1.wait()
    cp_w3.wait()
    cp_l.wait()
    pltpu.make_async_copy(bert_ref.at[pl.ds(0, B // 2)],
                          ebuf.at[pl.ds(0, B // 2)], sem_g).wait()
    pltpu.make_async_copy(bert_ref.at[pl.ds(0, B // 2)],
                          ebuf.at[pl.ds(0, B // 2)], sem_g2).wait()
    pltpu.make_async_copy(aux_ref.at[pl.ds(0, B)], pbuf, sem_p).wait()

    x = ebuf[...]
    cp_e = pltpu.make_async_copy(ebuf, emb_ref, sem_o)
    cp_e.start()
    h1 = 0.5 * jnp.tanh(
        0.5 * (jnp.dot(x, w1buf[...], preferred_element_type=jnp.float32)
               + b1_ref[...])) + 0.5
    a = jax.nn.sigmoid(
        jnp.sum(h1 * w2t_ref[...], axis=1, keepdims=True) + b2_ref[0, 0])
    h2 = 0.5 * jnp.tanh(
        0.5 * (jnp.dot(x, w3buf[...], preferred_element_type=jnp.float32)
               + b3_ref[...])) + 0.5
    bb = jnp.sum(h2 * w4t_ref[...], axis=1, keepdims=True) + b4_ref[0, 0]
    pcol = jnp.sum(pbuf[...] * lbuf[...], axis=1, keepdims=True)
    a8 = jnp.reshape(a, (8, 128))
    b8 = jnp.reshape(bb, (8, 128))
    p8 = jnp.reshape(pcol, (8, 128))
    prof_ref[...] = p8
    out_ref[...] = jax.nn.sigmoid(1.703 * a8 * (p8 - b8))
    cp_e.wait()


def kernel(stu_ids, exer_in, bert_table, stu_table,
           W_disc1, b_disc1, W_disc2, b_disc2,
           W_diff1, b_diff1, W_diff2, b_diff2):
    grid_spec = pltpu.PrefetchScalarGridSpec(
        num_scalar_prefetch=2,
        grid=(1,),
        in_specs=[
            pl.BlockSpec(memory_space=pl.ANY),
            pl.BlockSpec(memory_space=pl.ANY),
            pl.BlockSpec(memory_space=pl.ANY),
            pl.BlockSpec(memory_space=pl.ANY),
            pl.BlockSpec((1, H), lambda i, *_: (0, 0)),
            pl.BlockSpec((1, H), lambda i, *_: (0, 0)),
            pl.BlockSpec((1, D), lambda i, *_: (0, 0)),
            pl.BlockSpec((1, D), lambda i, *_: (0, 0)),
            pl.BlockSpec(memory_space=pltpu.SMEM),
            pl.BlockSpec(memory_space=pltpu.SMEM),
        ],
        out_specs=[
            pl.BlockSpec(memory_space=pl.ANY),
            pl.BlockSpec((8, 128), lambda i, *_: (0, 0)),
            pl.BlockSpec((8, 128), lambda i, *_: (0, 0)),
        ],
        scratch_shapes=[
            pltpu.VMEM((B, D), jnp.float32),
            pltpu.VMEM((B, 128), jnp.float32),
            pltpu.VMEM((B, 128), jnp.float32),
            pltpu.VMEM((D, H), jnp.float32),
            pltpu.VMEM((D, D), jnp.float32),
            pltpu.SemaphoreType.DMA,
            pltpu.SemaphoreType.DMA,
            pltpu.SemaphoreType.DMA,
            pltpu.SemaphoreType.DMA,
            pltpu.SemaphoreType.DMA,
        ],
    )
    sids32 = stu_ids.astype(jnp.int32)
    stu_rows = jnp.concatenate(
        [stu_table.reshape(-1),
         jnp.zeros((SROWS * 128 - 100000,), jnp.float32)]).reshape(SROWS, 128)
    onehot = (jnp.arange(128, dtype=jnp.int32)[None, :]
              == (sids32 & 127)[:, None]).astype(jnp.float32)
    aux = jnp.concatenate([stu_rows, onehot], axis=0)
    emb, prof, outc = pl.pallas_call(
        _mega_body,
        grid_spec=grid_spec,
        out_shape=[
            jax.ShapeDtypeStruct((B, D), jnp.float32),
            jax.ShapeDtypeStruct((8, 128), jnp.float32),
            jax.ShapeDtypeStruct((8, 128), jnp.float32),
        ],
    )(exer_in.astype(jnp.int32), sids32,
      bert_table, aux, W_disc1, W_diff1,
      b_disc1.reshape(1, H), W_disc2.reshape(1, H),
      b_diff1.reshape(1, D), W_diff2.reshape(1, D),
      b_disc2.reshape(1, 1), b_diff2.reshape(1, 1))
    return (outc.reshape(B), emb, prof.reshape(B, 1))
